# baseline (speedup 1.0000x reference)
import functools

import jax
import jax.numpy as jnp
from jax import lax
from jax.experimental import pallas as pl
from jax.experimental.pallas import tpu as pltpu
from jax.experimental.pallas import tpu_sc as plsc

_N = 16384
_D = 128
_K = 64
_L = 16
_NW = 32
_PW = _N // _NW
_TW = 128
_C = 256
_NCHUNK = _PW // _C


def _body(inp1_hbm, idx_hbm, out_hbm, table_v, idx_v, block_v, out_v):
    wid = lax.axis_index("s") * 2 + lax.axis_index("c")
    base = wid * _PW

    pltpu.sync_copy(inp1_hbm.at[:, pl.ds(0, _TW)], table_v)
    pltpu.sync_copy(idx_hbm.at[pl.ds(base, _PW)], idx_v)

    iota = lax.iota(jnp.int32, _L)

    for c in range(_NCHUNK):
        cbase = base + c * _C
        pltpu.sync_copy(inp1_hbm.at[:, pl.ds(cbase, _C)], block_v)

        def group(gi, _, c=c):
            i0 = gi * _L
            idx_vec = idx_v[pl.ds(c * _C + i0, _L)]
            idx_vec = jnp.clip(idx_vec, 0, _K - 1)
            rows = iota + i0

            def jloop(j, __):
                jvec = jnp.full((_L,), j, jnp.int32)
                a = block_v[j, pl.ds(i0, _L)]
                t = plsc.load_gather(table_v, [jvec, idx_vec])
                plsc.store_scatter(out_v, [rows, jvec], a * t * t)
                return __

            lax.fori_loop(0, _D, jloop, 0)
            return _

        lax.fori_loop(0, _C // _L, group, 0)
        pltpu.sync_copy(out_v, out_hbm.at[pl.ds(cbase, _C), :])


def kernel(inp1, inp2):
    idx = inp2.reshape(-1).astype(jnp.int32)
    mesh = plsc.VectorSubcoreMesh(core_axis_name="c", subcore_axis_name="s")
    k = functools.partial(
        pl.kernel,
        mesh=mesh,
        out_type=jax.ShapeDtypeStruct((_N, _D), jnp.float32),
        scratch_types=[
            pltpu.VMEM((_D, _TW), jnp.float32),
            pltpu.VMEM((_PW,), jnp.int32),
            pltpu.VMEM((_D, _C), jnp.float32),
            pltpu.VMEM((_C, _D), jnp.float32),
        ],
        compiler_params=pltpu.CompilerParams(needs_layout_passes=False),
    )(_body)
    return (k(inp1, idx),)
